# Initial kernel scaffold; baseline (speedup 1.0000x reference)
import jax
import jax.numpy as jnp
from jax.experimental import pallas as pl


def kernel(loss, type_mask):
    raise NotImplementedError("write your pallas kernel here")


# speedup vs baseline: 41.5952x; 41.5952x over previous
import functools

import numpy as np

import jax
import jax.numpy as jnp
from jax import lax
from jax.experimental import pallas as pl
from jax.experimental.pallas import tpu as pltpu
from jax.experimental.pallas import tpu_sc as plsc

N, C, HW = 16, 8, 256 * 256
M = N * C * HW
L = N * HW
NCORES, NSUB, LN = 2, 16, 16
NW = NCORES * NSUB
SLABS_PER_TILE = 4
CH = 8192
CPS = HW // CH
NCHUNK = SLABS_PER_TILE * CPS
NB1, NB2, NB3 = 4096, 4096, 64
SW = 4 * LN
IMIN = np.int32(-(2 ** 31))
IMAX = np.int32(2 ** 31 - 1)

_mesh = plsc.VectorSubcoreMesh(core_axis_name="c", subcore_axis_name="s")


def _ids():
    ci = lax.axis_index("c")
    si = lax.axis_index("s")
    wid = ci * NSUB + si
    g = ci * 4 + si // 4
    st = si % 4
    return wid, g, st


def _zero_1d(ref, n):
    z = jnp.zeros((LN,), jnp.int32)

    def body(j, _):
        ref[pl.ds(j * LN, LN)] = z
        return 0

    lax.fori_loop(0, n // LN, body, 0)


def _merge_lanes(hist, hsum, nb):
    def col(j, _):
        def row(l, a):
            return a + hist[pl.ds(l * nb + j * LN, LN)]
        v = lax.fori_loop(0, LN, row, jnp.zeros((LN,), jnp.int32))
        hsum[pl.ds(j * LN, LN)] = v
        return 0

    lax.fori_loop(0, nb // LN, col, 0)


def _quad_sum(h4, hsum, nb):
    def col(j, acc):
        v = (h4[pl.ds(j * LN, LN)] + h4[pl.ds(nb + j * LN, LN)]
             + h4[pl.ds(2 * nb + j * LN, LN)] + h4[pl.ds(3 * nb + j * LN, LN)])
        hsum[pl.ds(j * LN, LN)] = v
        return acc + v

    tot = lax.fori_loop(0, nb // LN, col, jnp.zeros((LN,), jnp.int32))
    return jnp.sum(tot)


def _scan_threshold(hsum, nb, target):
    iota = lax.iota(jnp.int32, LN)

    def blk(j, carry):
        found, bsel, csel, run = carry
        v = hsum[pl.ds(j * LN, LN)]
        pref = plsc.cumsum(v)
        cvec = pref + run
        m = cvec >= target
        mi = m.astype(jnp.int32)
        blockhit = jnp.max(mi)
        lane = jnp.min(jnp.where(m, iota, jnp.int32(LN)))
        cblk = jnp.min(jnp.where(m, cvec, IMAX))
        take = (found == 0) & (blockhit == 1)
        bsel = jnp.where(take, j * LN + lane, bsel)
        csel = jnp.where(take, cblk, csel)
        found = jnp.maximum(found, blockhit)
        run = run + jnp.sum(v)
        return found, bsel, csel, run

    found, bsel, csel, _ = lax.fori_loop(
        0, nb // LN, blk,
        (jnp.int32(0), jnp.int32(-1), jnp.int32(0), jnp.int32(0)))
    return bsel, csel


def _splat(x):
    return jnp.full((LN,), x, jnp.int32)


def _read_scalar(vec_ref, row):
    return jnp.max(vec_ref[pl.ds(row * LN, LN)])


def _chunk_base(g, st, t):
    slab = st * SLABS_PER_TILE + t // CPS
    return (slab * C + g) * HW + (t % CPS) * CH


@functools.partial(
    pl.kernel,
    out_type=[jax.ShapeDtypeStruct((M,), jnp.int32),
              jax.ShapeDtypeStruct((NW * NB1,), jnp.int32),
              jax.ShapeDtypeStruct((NW * LN,), jnp.int32)],
    mesh=_mesh,
    compiler_params=pltpu.CompilerParams(needs_layout_passes=False),
    scratch_types=[pltpu.VMEM((CH,), jnp.float32),
                   pltpu.VMEM((CH,), jnp.int32),
                   pltpu.VMEM((CH,), jnp.int32),
                   pltpu.VMEM((LN * NB1,), jnp.int32),
                   pltpu.VMEM((NB1,), jnp.int32),
                   pltpu.VMEM((LN,), jnp.int32)],
)
def _pass_a(loss_hbm, tm_hbm, kaug_hbm, hist_hbm, cnt_hbm,
            lbuf, mbuf, kbuf, hist, hsum, cnt):
    wid, g, st = _ids()
    lanes = lax.iota(jnp.int32, LN)
    lanebase = lanes * NB1
    ones = jnp.ones((LN,), jnp.int32)
    zeros = jnp.zeros((LN,), jnp.int32)

    _zero_1d(hist, LN * NB1)
    cnt[...] = zeros

    def chunk(t, _):
        base = _chunk_base(g, st, t)
        pltpu.sync_copy(loss_hbm.at[pl.ds(base, CH)], lbuf)
        pltpu.sync_copy(tm_hbm.at[pl.ds(base, CH)], mbuf)

        def vec(i, _):
            lv = lbuf[pl.ds(i * LN, LN)]
            tm = mbuf[pl.ds(i * LN, LN)]
            masked = jnp.where(tm == 0, lv, jnp.float32(0.0))
            key = lax.bitcast_convert_type(masked, jnp.int32)
            pos = tm == 1
            kbuf[pl.ds(i * LN, LN)] = jnp.where(pos, IMIN, key)
            bin1 = lax.shift_right_logical(key, 18)
            plsc.addupdate_scatter(hist.at[:], [lanebase + bin1], ones,
                                   mask=key != 0)
            cnt[...] = cnt[...] + jnp.where(pos, ones, zeros)
            return 0

        lax.fori_loop(0, CH // LN, vec, 0)
        pltpu.sync_copy(kbuf, kaug_hbm.at[pl.ds(base, CH)])
        return 0

    lax.fori_loop(0, NCHUNK, chunk, 0)
    _merge_lanes(hist, hsum, NB1)
    pltpu.sync_copy(hsum, hist_hbm.at[pl.ds(wid * NB1, NB1)])
    pltpu.sync_copy(cnt, cnt_hbm.at[pl.ds(wid * LN, LN)])


@functools.partial(
    pl.kernel,
    out_type=[jax.ShapeDtypeStruct((NW * NB2,), jnp.int32),
              jax.ShapeDtypeStruct((NW * SW,), jnp.int32)],
    mesh=_mesh,
    compiler_params=pltpu.CompilerParams(needs_layout_passes=False),
    scratch_types=[pltpu.VMEM((CH,), jnp.int32),
                   pltpu.VMEM((LN * NB2,), jnp.int32),
                   pltpu.VMEM((4 * NB1,), jnp.int32),
                   pltpu.VMEM((NB1,), jnp.int32),
                   pltpu.VMEM((4 * LN,), jnp.int32),
                   pltpu.VMEM((SW,), jnp.int32)],
)
def _pass_c(kaug_hbm, hist1_hbm, cnt_hbm, hist2_hbm, st1_hbm,
            kbuf, hist, h4, hsum, c4, stage):
    wid, g, st = _ids()
    qbase = (wid // 4) * 4
    lanes = lax.iota(jnp.int32, LN)
    lanebase = lanes * NB2
    ones = jnp.ones((LN,), jnp.int32)

    pltpu.sync_copy(hist1_hbm.at[pl.ds(qbase * NB1, 4 * NB1)], h4)
    pltpu.sync_copy(cnt_hbm.at[pl.ds(qbase * LN, 4 * LN)], c4)
    total = _quad_sum(h4, hsum, NB1)
    cntpos = jnp.sum(c4[pl.ds(0, LN)] + c4[pl.ds(LN, LN)]
                     + c4[pl.ds(2 * LN, LN)] + c4[pl.ds(3 * LN, LN)])
    k = jnp.minimum(jnp.maximum(cntpos, 1), total)
    target = total - k + 1
    b1, cb1 = _scan_threshold(hsum, NB1, target)
    r1 = k - (total - cb1)
    valid = (total > 0).astype(jnp.int32)

    stage[pl.ds(0, LN)] = _splat(b1)
    stage[pl.ds(LN, LN)] = _splat(r1)
    stage[pl.ds(2 * LN, LN)] = _splat(valid)
    stage[pl.ds(3 * LN, LN)] = _splat(k)
    pltpu.sync_copy(stage, st1_hbm.at[pl.ds(wid * SW, SW)])

    _zero_1d(hist, LN * NB2)
    b1v = _splat(b1)

    def chunk(t, _):
        base = _chunk_base(g, st, t)
        pltpu.sync_copy(kaug_hbm.at[pl.ds(base, CH)], kbuf)

        def vec(i, _):
            kaug = kbuf[pl.ds(i * LN, LN)]
            m = (lax.shift_right_logical(kaug, 18) == b1v) & (kaug != 0)
            bin2 = lax.shift_right_logical(kaug, 6) & jnp.int32(0xFFF)
            plsc.addupdate_scatter(hist.at[:], [lanebase + bin2], ones, mask=m)
            return 0

        lax.fori_loop(0, CH // LN, vec, 0)
        return 0

    lax.fori_loop(0, NCHUNK, chunk, 0)
    _merge_lanes(hist, hsum, NB2)
    pltpu.sync_copy(hsum, hist2_hbm.at[pl.ds(wid * NB2, NB2)])


@functools.partial(
    pl.kernel,
    out_type=[jax.ShapeDtypeStruct((NW * NB3,), jnp.int32),
              jax.ShapeDtypeStruct((NW * SW,), jnp.int32)],
    mesh=_mesh,
    compiler_params=pltpu.CompilerParams(needs_layout_passes=False),
    scratch_types=[pltpu.VMEM((CH,), jnp.int32),
                   pltpu.VMEM((LN * NB3,), jnp.int32),
                   pltpu.VMEM((4 * NB2,), jnp.int32),
                   pltpu.VMEM((NB2,), jnp.int32),
                   pltpu.VMEM((SW,), jnp.int32),
                   pltpu.VMEM((SW,), jnp.int32)],
)
def _pass_e(kaug_hbm, hist2_hbm, st1_hbm, hist3_hbm, st2_hbm,
            kbuf, hist, h4, hsum, st1v, stage):
    wid, g, st = _ids()
    qbase = (wid // 4) * 4
    lanes = lax.iota(jnp.int32, LN)
    lanebase = lanes * NB3
    ones = jnp.ones((LN,), jnp.int32)

    pltpu.sync_copy(st1_hbm.at[pl.ds(wid * SW, SW)], st1v)
    b1 = _read_scalar(st1v, 0)
    r1 = _read_scalar(st1v, 1)
    valid = _read_scalar(st1v, 2)

    pltpu.sync_copy(hist2_hbm.at[pl.ds(qbase * NB2, 4 * NB2)], h4)
    total2 = _quad_sum(h4, hsum, NB2)
    target2 = total2 - r1 + 1
    b2, cb2 = _scan_threshold(hsum, NB2, target2)
    r2 = r1 - (total2 - cb2)
    b12 = lax.shift_left(b1, 12) | b2

    stage[pl.ds(0, LN)] = _splat(b12)
    stage[pl.ds(LN, LN)] = _splat(r2)
    stage[pl.ds(2 * LN, LN)] = _splat(valid)
    stage[pl.ds(3 * LN, LN)] = _splat(0)
    pltpu.sync_copy(stage, st2_hbm.at[pl.ds(wid * SW, SW)])

    _zero_1d(hist, LN * NB3)
    b12v = _splat(b12)

    def chunk(t, _):
        base = _chunk_base(g, st, t)
        pltpu.sync_copy(kaug_hbm.at[pl.ds(base, CH)], kbuf)

        def vec(i, _):
            kaug = kbuf[pl.ds(i * LN, LN)]
            m = (lax.shift_right_logical(kaug, 6) == b12v) & (kaug != 0)
            bin3 = kaug & jnp.int32(0x3F)
            plsc.addupdate_scatter(hist.at[:], [lanebase + bin3], ones, mask=m)
            return 0

        lax.fori_loop(0, CH // LN, vec, 0)
        return 0

    lax.fori_loop(0, NCHUNK, chunk, 0)
    _merge_lanes(hist, hsum, NB3)
    pltpu.sync_copy(hsum.at[pl.ds(0, NB3)], hist3_hbm.at[pl.ds(wid * NB3, NB3)])


@functools.partial(
    pl.kernel,
    out_type=jax.ShapeDtypeStruct((M,), jnp.float32),
    mesh=_mesh,
    compiler_params=pltpu.CompilerParams(needs_layout_passes=False),
    scratch_types=[pltpu.VMEM((CH,), jnp.int32),
                   pltpu.VMEM((CH,), jnp.float32),
                   pltpu.VMEM((4 * NB3,), jnp.int32),
                   pltpu.VMEM((NB3,), jnp.int32),
                   pltpu.VMEM((SW,), jnp.int32)],
)
def _pass_g(kaug_hbm, hist3_hbm, st2_hbm, out_hbm, kbuf, obuf, h4, hsum, st2v):
    wid, g, st = _ids()
    qbase = (wid // 4) * 4

    pltpu.sync_copy(st2_hbm.at[pl.ds(wid * SW, SW)], st2v)
    b12 = _read_scalar(st2v, 0)
    r2 = _read_scalar(st2v, 1)
    valid = _read_scalar(st2v, 2)

    pltpu.sync_copy(hist3_hbm.at[pl.ds(qbase * NB3, 4 * NB3)], h4)
    total3 = _quad_sum(h4, hsum, NB3)
    target3 = total3 - r2 + 1
    b3, _cb3 = _scan_threshold(hsum, NB3, target3)
    t = lax.shift_left(b12, 6) | b3
    t = jnp.where(valid == 1, t, jnp.int32(2 ** 30))
    txor = _splat(t ^ IMIN)

    def chunk(tt, _):
        base = _chunk_base(g, st, tt)
        pltpu.sync_copy(kaug_hbm.at[pl.ds(base, CH)], kbuf)

        def vec(i, _):
            kaug = kbuf[pl.ds(i * LN, LN)]
            sel = (kaug ^ IMIN) >= txor
            obuf[pl.ds(i * LN, LN)] = jnp.where(sel, jnp.float32(1.0),
                                                jnp.float32(0.0))
            return 0

        lax.fori_loop(0, CH // LN, vec, 0)
        pltpu.sync_copy(obuf, out_hbm.at[pl.ds(base, CH)])
        return 0

    lax.fori_loop(0, NCHUNK, chunk, 0)


def kernel(loss, type_mask):
    lf = loss.reshape(M)
    tf = type_mask.reshape(M)
    kaug, hist1, cnt = _pass_a(lf, tf)
    hist2, st1 = _pass_c(kaug, hist1, cnt)
    hist3, st2 = _pass_e(kaug, hist2, st1)
    out = _pass_g(kaug, hist3, st2)
    return out.reshape(loss.shape)


# speedup vs baseline: 65.6648x; 1.5787x over previous
import functools

import numpy as np

import jax
import jax.numpy as jnp
from jax import lax
from jax.experimental import pallas as pl
from jax.experimental.pallas import tpu as pltpu
from jax.experimental.pallas import tpu_sc as plsc

N, C, HW = 16, 8, 256 * 256
M = N * C * HW
L = N * HW
NCORES, NSUB, LN = 2, 16, 16
NW = NCORES * NSUB
SLABS_PER_TILE = 4
CH = 8192
CPS = HW // CH
NCHUNK = SLABS_PER_TILE * CPS
NB1, NB2, NB3 = 4096, 4096, 64
SW = 4 * LN
U = 8
IMIN = np.int32(-(2 ** 31))
IMAX = np.int32(2 ** 31 - 1)

_mesh = plsc.VectorSubcoreMesh(core_axis_name="c", subcore_axis_name="s")


def _ids():
    ci = lax.axis_index("c")
    si = lax.axis_index("s")
    wid = ci * NSUB + si
    g = ci * 4 + si // 4
    st = si % 4
    return wid, g, st


def _zero_1d(ref, n):
    z = jnp.zeros((LN,), jnp.int32)

    def body(j, _):
        for u in range(U):
            ref[pl.ds((j * U + u) * LN, LN)] = z
        return 0

    lax.fori_loop(0, n // (LN * U), body, 0)


def _merge_lanes(hist, hsum, nb):
    def col(j, _):
        v = hist[pl.ds(j * LN, LN)]
        for l in range(1, LN):
            v = v + hist[pl.ds(l * nb + j * LN, LN)]
        hsum[pl.ds(j * LN, LN)] = v
        return 0

    lax.fori_loop(0, nb // LN, col, 0)


def _quad_sum(h4, hsum, nb):
    def col(j, acc):
        v = (h4[pl.ds(j * LN, LN)] + h4[pl.ds(nb + j * LN, LN)]
             + h4[pl.ds(2 * nb + j * LN, LN)] + h4[pl.ds(3 * nb + j * LN, LN)])
        hsum[pl.ds(j * LN, LN)] = v
        return acc + v

    tot = lax.fori_loop(0, nb // LN, col, jnp.zeros((LN,), jnp.int32))
    return jnp.sum(tot)


def _scan_threshold(hsum, nb, target):
    iota = lax.iota(jnp.int32, LN)

    def blk(j, carry):
        found, bsel, csel, run = carry
        v = hsum[pl.ds(j * LN, LN)]
        pref = plsc.cumsum(v)
        cvec = pref + run
        m = cvec >= target
        mi = m.astype(jnp.int32)
        blockhit = jnp.max(mi)
        lane = jnp.min(jnp.where(m, iota, jnp.int32(LN)))
        cblk = jnp.min(jnp.where(m, cvec, IMAX))
        take = (found == 0) & (blockhit == 1)
        bsel = jnp.where(take, j * LN + lane, bsel)
        csel = jnp.where(take, cblk, csel)
        found = jnp.maximum(found, blockhit)
        run = run + jnp.sum(v)
        return found, bsel, csel, run

    found, bsel, csel, _ = lax.fori_loop(
        0, nb // LN, blk,
        (jnp.int32(0), jnp.int32(-1), jnp.int32(0), jnp.int32(0)))
    return bsel, csel


def _splat(x):
    return jnp.full((LN,), x, jnp.int32)


def _read_scalar(vec_ref, row):
    return jnp.max(vec_ref[pl.ds(row * LN, LN)])


def _chunk_base(g, st, t):
    slab = st * SLABS_PER_TILE + t // CPS
    return (slab * C + g) * HW + (t % CPS) * CH


@functools.partial(
    pl.kernel,
    out_type=[jax.ShapeDtypeStruct((M,), jnp.int32),
              jax.ShapeDtypeStruct((NW * NB1,), jnp.int32),
              jax.ShapeDtypeStruct((NW * LN,), jnp.int32)],
    mesh=_mesh,
    compiler_params=pltpu.CompilerParams(needs_layout_passes=False),
    scratch_types=[pltpu.VMEM((CH,), jnp.float32),
                   pltpu.VMEM((CH,), jnp.float32),
                   pltpu.VMEM((CH,), jnp.int32),
                   pltpu.VMEM((CH,), jnp.int32),
                   pltpu.VMEM((CH,), jnp.int32),
                   pltpu.VMEM((CH,), jnp.int32),
                   pltpu.VMEM((LN * NB1,), jnp.int32),
                   pltpu.VMEM((NB1,), jnp.int32),
                   pltpu.VMEM((LN,), jnp.int32),
                   pltpu.SemaphoreType.DMA,
                   pltpu.SemaphoreType.DMA,
                   pltpu.SemaphoreType.DMA,
                   pltpu.SemaphoreType.DMA,
                   pltpu.SemaphoreType.DMA,
                   pltpu.SemaphoreType.DMA],
)
def _pass_a(loss_hbm, tm_hbm, kaug_hbm, hist_hbm, cnt_hbm,
            lbuf0, lbuf1, mbuf0, mbuf1, kbuf0, kbuf1, hist, hsum, cnt,
            sl0, sl1, sm0, sm1, so0, so1):
    wid, g, st = _ids()
    lanes = lax.iota(jnp.int32, LN)
    lanebase = lanes * NB1
    ones = jnp.ones((LN,), jnp.int32)
    zeros = jnp.zeros((LN,), jnp.int32)

    lbufs, mbufs, kbufs = (lbuf0, lbuf1), (mbuf0, mbuf1), (kbuf0, kbuf1)
    sls, sms, sos = (sl0, sl1), (sm0, sm1), (so0, so1)

    def start_in(t, s):
        base = _chunk_base(g, st, t)
        pltpu.async_copy(loss_hbm.at[pl.ds(base, CH)], lbufs[s], sls[s])
        pltpu.async_copy(tm_hbm.at[pl.ds(base, CH)], mbufs[s], sms[s])

    def wait_in(t, s):
        base = _chunk_base(g, st, t)
        pltpu.make_async_copy(loss_hbm.at[pl.ds(base, CH)], lbufs[s], sls[s]).wait()
        pltpu.make_async_copy(tm_hbm.at[pl.ds(base, CH)], mbufs[s], sms[s]).wait()

    def start_out(t, s):
        base = _chunk_base(g, st, t)
        pltpu.async_copy(kbufs[s], kaug_hbm.at[pl.ds(base, CH)], sos[s])

    def wait_out(t, s):
        base = _chunk_base(g, st, t)
        pltpu.make_async_copy(kbufs[s], kaug_hbm.at[pl.ds(base, CH)], sos[s]).wait()

    def compute(s):
        lbuf, mbuf, kbuf = lbufs[s], mbufs[s], kbufs[s]

        def vec(i, _):
            csum = None
            for u in range(U):
                off = (i * U + u) * LN
                lv = lbuf[pl.ds(off, LN)]
                tm = mbuf[pl.ds(off, LN)]
                masked = jnp.where(tm == 0, lv, jnp.float32(0.0))
                key = lax.bitcast_convert_type(masked, jnp.int32)
                kbuf[pl.ds(off, LN)] = jnp.where(tm == 1, IMIN, key)
                bin1 = lax.shift_right_logical(key, 18)
                plsc.addupdate_scatter(hist.at[:], [lanebase + bin1], ones,
                                       mask=key != 0)
                csum = tm if csum is None else csum + tm
            cnt[...] = cnt[...] + csum
            return 0

        lax.fori_loop(0, CH // (LN * U), vec, 0)

    _zero_1d(hist, LN * NB1)
    cnt[...] = zeros

    start_in(0, 0)
    start_in(1, 1)

    def pair(i2, _):
        for s in range(2):
            t = 2 * i2 + s

            wait_in(t, s)

            @pl.when(t >= 2)
            def _():
                wait_out(t - 2, s)

            compute(s)
            start_out(t, s)

            @pl.when(t + 2 < NCHUNK)
            def _():
                start_in(t + 2, s)
        return 0

    lax.fori_loop(0, NCHUNK // 2, pair, 0)
    wait_out(NCHUNK - 2, 0)
    wait_out(NCHUNK - 1, 1)

    _merge_lanes(hist, hsum, NB1)
    pltpu.sync_copy(hsum, hist_hbm.at[pl.ds(wid * NB1, NB1)])
    pltpu.sync_copy(cnt, cnt_hbm.at[pl.ds(wid * LN, LN)])


def _refine_stream(kaug_hbm, g, st, kbufs, sems, vec_fn):
    def start_in(t, s):
        base = _chunk_base(g, st, t)
        pltpu.async_copy(kaug_hbm.at[pl.ds(base, CH)], kbufs[s], sems[s])

    def wait_in(t, s):
        base = _chunk_base(g, st, t)
        pltpu.make_async_copy(kaug_hbm.at[pl.ds(base, CH)], kbufs[s],
                              sems[s]).wait()

    start_in(0, 0)
    start_in(1, 1)

    def pair(i2, _):
        for s in range(2):
            t = 2 * i2 + s
            wait_in(t, s)

            def vec(i, _):
                for u in range(U):
                    vec_fn(kbufs[s], (i * U + u) * LN)
                return 0

            lax.fori_loop(0, CH // (LN * U), vec, 0)

            @pl.when(t + 2 < NCHUNK)
            def _():
                start_in(t + 2, s)
        return 0

    lax.fori_loop(0, NCHUNK // 2, pair, 0)


@functools.partial(
    pl.kernel,
    out_type=[jax.ShapeDtypeStruct((NW * NB2,), jnp.int32),
              jax.ShapeDtypeStruct((NW * SW,), jnp.int32)],
    mesh=_mesh,
    compiler_params=pltpu.CompilerParams(needs_layout_passes=False),
    scratch_types=[pltpu.VMEM((CH,), jnp.int32),
                   pltpu.VMEM((CH,), jnp.int32),
                   pltpu.VMEM((LN * NB2,), jnp.int32),
                   pltpu.VMEM((4 * NB1,), jnp.int32),
                   pltpu.VMEM((NB1,), jnp.int32),
                   pltpu.VMEM((4 * LN,), jnp.int32),
                   pltpu.VMEM((SW,), jnp.int32),
                   pltpu.SemaphoreType.DMA,
                   pltpu.SemaphoreType.DMA],
)
def _pass_c(kaug_hbm, hist1_hbm, cnt_hbm, hist2_hbm, st1_hbm,
            kbuf0, kbuf1, hist, h4, hsum, c4, stage, s0, s1):
    wid, g, st = _ids()
    qbase = (wid // 4) * 4
    lanes = lax.iota(jnp.int32, LN)
    lanebase = lanes * NB2
    ones = jnp.ones((LN,), jnp.int32)

    _zero_1d(hist, LN * NB2)

    pltpu.sync_copy(hist1_hbm.at[pl.ds(qbase * NB1, 4 * NB1)], h4)
    pltpu.sync_copy(cnt_hbm.at[pl.ds(qbase * LN, 4 * LN)], c4)
    total = _quad_sum(h4, hsum, NB1)
    cntpos = jnp.sum(c4[pl.ds(0, LN)] + c4[pl.ds(LN, LN)]
                     + c4[pl.ds(2 * LN, LN)] + c4[pl.ds(3 * LN, LN)])
    k = jnp.minimum(jnp.maximum(cntpos, 1), total)
    target = total - k + 1
    b1, cb1 = _scan_threshold(hsum, NB1, target)
    r1 = k - (total - cb1)
    valid = (total > 0).astype(jnp.int32)

    stage[pl.ds(0, LN)] = _splat(b1)
    stage[pl.ds(LN, LN)] = _splat(r1)
    stage[pl.ds(2 * LN, LN)] = _splat(valid)
    stage[pl.ds(3 * LN, LN)] = _splat(k)
    pltpu.sync_copy(stage, st1_hbm.at[pl.ds(wid * SW, SW)])

    b1v = _splat(b1)

    def vec_fn(kbuf, off):
        kaug = kbuf[pl.ds(off, LN)]
        m = (lax.shift_right_logical(kaug, 18) == b1v) & (kaug != 0)
        bin2 = lax.shift_right_logical(kaug, 6) & jnp.int32(0xFFF)
        plsc.addupdate_scatter(hist.at[:], [lanebase + bin2], ones, mask=m)

    _refine_stream(kaug_hbm, g, st, (kbuf0, kbuf1), (s0, s1), vec_fn)

    _merge_lanes(hist, hsum, NB2)
    pltpu.sync_copy(hsum, hist2_hbm.at[pl.ds(wid * NB2, NB2)])


@functools.partial(
    pl.kernel,
    out_type=[jax.ShapeDtypeStruct((NW * NB3,), jnp.int32),
              jax.ShapeDtypeStruct((NW * SW,), jnp.int32)],
    mesh=_mesh,
    compiler_params=pltpu.CompilerParams(needs_layout_passes=False),
    scratch_types=[pltpu.VMEM((CH,), jnp.int32),
                   pltpu.VMEM((CH,), jnp.int32),
                   pltpu.VMEM((LN * NB3,), jnp.int32),
                   pltpu.VMEM((4 * NB2,), jnp.int32),
                   pltpu.VMEM((NB2,), jnp.int32),
                   pltpu.VMEM((SW,), jnp.int32),
                   pltpu.VMEM((SW,), jnp.int32),
                   pltpu.SemaphoreType.DMA,
                   pltpu.SemaphoreType.DMA],
)
def _pass_e(kaug_hbm, hist2_hbm, st1_hbm, hist3_hbm, st2_hbm,
            kbuf0, kbuf1, hist, h4, hsum, st1v, stage, s0, s1):
    wid, g, st = _ids()
    qbase = (wid // 4) * 4
    lanes = lax.iota(jnp.int32, LN)
    lanebase = lanes * NB3
    ones = jnp.ones((LN,), jnp.int32)

    _zero_1d(hist, LN * NB3)

    pltpu.sync_copy(st1_hbm.at[pl.ds(wid * SW, SW)], st1v)
    b1 = _read_scalar(st1v, 0)
    r1 = _read_scalar(st1v, 1)
    valid = _read_scalar(st1v, 2)

    pltpu.sync_copy(hist2_hbm.at[pl.ds(qbase * NB2, 4 * NB2)], h4)
    total2 = _quad_sum(h4, hsum, NB2)
    target2 = total2 - r1 + 1
    b2, cb2 = _scan_threshold(hsum, NB2, target2)
    r2 = r1 - (total2 - cb2)
    b12 = lax.shift_left(b1, 12) | b2

    stage[pl.ds(0, LN)] = _splat(b12)
    stage[pl.ds(LN, LN)] = _splat(r2)
    stage[pl.ds(2 * LN, LN)] = _splat(valid)
    stage[pl.ds(3 * LN, LN)] = _splat(0)
    pltpu.sync_copy(stage, st2_hbm.at[pl.ds(wid * SW, SW)])

    b12v = _splat(b12)

    def vec_fn(kbuf, off):
        kaug = kbuf[pl.ds(off, LN)]
        m = (lax.shift_right_logical(kaug, 6) == b12v) & (kaug != 0)
        bin3 = kaug & jnp.int32(0x3F)
        plsc.addupdate_scatter(hist.at[:], [lanebase + bin3], ones, mask=m)

    _refine_stream(kaug_hbm, g, st, (kbuf0, kbuf1), (s0, s1), vec_fn)

    _merge_lanes(hist, hsum, NB3)
    pltpu.sync_copy(hsum.at[pl.ds(0, NB3)], hist3_hbm.at[pl.ds(wid * NB3, NB3)])


@functools.partial(
    pl.kernel,
    out_type=jax.ShapeDtypeStruct((M,), jnp.float32),
    mesh=_mesh,
    compiler_params=pltpu.CompilerParams(needs_layout_passes=False),
    scratch_types=[pltpu.VMEM((CH,), jnp.int32),
                   pltpu.VMEM((CH,), jnp.int32),
                   pltpu.VMEM((CH,), jnp.float32),
                   pltpu.VMEM((CH,), jnp.float32),
                   pltpu.VMEM((4 * NB3,), jnp.int32),
                   pltpu.VMEM((NB3,), jnp.int32),
                   pltpu.VMEM((SW,), jnp.int32),
                   pltpu.SemaphoreType.DMA,
                   pltpu.SemaphoreType.DMA,
                   pltpu.SemaphoreType.DMA,
                   pltpu.SemaphoreType.DMA],
)
def _pass_g(kaug_hbm, hist3_hbm, st2_hbm, out_hbm,
            kbuf0, kbuf1, obuf0, obuf1, h4, hsum, st2v, si0, si1, so0, so1):
    wid, g, st = _ids()
    qbase = (wid // 4) * 4

    pltpu.sync_copy(st2_hbm.at[pl.ds(wid * SW, SW)], st2v)
    b12 = _read_scalar(st2v, 0)
    r2 = _read_scalar(st2v, 1)
    valid = _read_scalar(st2v, 2)

    pltpu.sync_copy(hist3_hbm.at[pl.ds(qbase * NB3, 4 * NB3)], h4)
    total3 = _quad_sum(h4, hsum, NB3)
    target3 = total3 - r2 + 1
    b3, _cb3 = _scan_threshold(hsum, NB3, target3)
    t = lax.shift_left(b12, 6) | b3
    t = jnp.where(valid == 1, t, jnp.int32(2 ** 30))
    txor = _splat(t ^ IMIN)

    kbufs, obufs = (kbuf0, kbuf1), (obuf0, obuf1)
    sis, sos = (si0, si1), (so0, so1)

    def start_in(t_, s):
        base = _chunk_base(g, st, t_)
        pltpu.async_copy(kaug_hbm.at[pl.ds(base, CH)], kbufs[s], sis[s])

    def wait_in(t_, s):
        base = _chunk_base(g, st, t_)
        pltpu.make_async_copy(kaug_hbm.at[pl.ds(base, CH)], kbufs[s],
                              sis[s]).wait()

    def start_out(t_, s):
        base = _chunk_base(g, st, t_)
        pltpu.async_copy(obufs[s], out_hbm.at[pl.ds(base, CH)], sos[s])

    def wait_out(t_, s):
        base = _chunk_base(g, st, t_)
        pltpu.make_async_copy(obufs[s], out_hbm.at[pl.ds(base, CH)],
                              sos[s]).wait()

    start_in(0, 0)
    start_in(1, 1)

    def pair(i2, _):
        for s in range(2):
            t_ = 2 * i2 + s
            wait_in(t_, s)

            @pl.when(t_ >= 2)
            def _():
                wait_out(t_ - 2, s)

            obuf, kbuf = obufs[s], kbufs[s]

            def vec(i, _):
                for u in range(U):
                    off = (i * U + u) * LN
                    kaug = kbuf[pl.ds(off, LN)]
                    sel = (kaug ^ IMIN) >= txor
                    obuf[pl.ds(off, LN)] = jnp.where(sel, jnp.float32(1.0),
                                                     jnp.float32(0.0))
                return 0

            lax.fori_loop(0, CH // (LN * U), vec, 0)
            start_out(t_, s)

            @pl.when(t_ + 2 < NCHUNK)
            def _():
                start_in(t_ + 2, s)
        return 0

    lax.fori_loop(0, NCHUNK // 2, pair, 0)
    wait_out(NCHUNK - 2, 0)
    wait_out(NCHUNK - 1, 1)


def kernel(loss, type_mask):
    lf = loss.reshape(M)
    tf = type_mask.reshape(M)
    kaug, hist1, cnt = _pass_a(lf, tf)
    hist2, st1 = _pass_c(kaug, hist1, cnt)
    hist3, st2 = _pass_e(kaug, hist2, st1)
    out = _pass_g(kaug, hist3, st2)
    return out.reshape(loss.shape)


# speedup vs baseline: 152.6709x; 2.3250x over previous
import functools

import numpy as np

import jax
import jax.numpy as jnp
from jax import lax
from jax.experimental import pallas as pl
from jax.experimental.pallas import tpu as pltpu
from jax.experimental.pallas import tpu_sc as plsc

N, C, HW = 16, 8, 256 * 256
M = N * C * HW
L = N * HW
NCORES, NSUB, LN = 2, 16, 16
NW = NCORES * NSUB
SLABS_PER_TILE = 4
CH = 8192
CPS = HW // CH
NCHUNK = SLABS_PER_TILE * CPS
NB1, NB2, NB3 = 4096, 4096, 64
U = 8
IMIN = np.int32(-(2 ** 31))
IMAX = np.int32(2 ** 31 - 1)

_mesh = plsc.VectorSubcoreMesh(core_axis_name="c", subcore_axis_name="s")


def _zero_1d(ref, n):
    z = jnp.zeros((LN,), jnp.int32)

    def body(j, _):
        for u in range(U):
            ref[pl.ds((j * U + u) * LN, LN)] = z
        return 0

    lax.fori_loop(0, n // (LN * U), body, 0)


def _merge_lanes(hist, hsum, nb):
    def col(j, _):
        v = hist[pl.ds(j * LN, LN)]
        for l in range(1, LN):
            v = v + hist[pl.ds(l * nb + j * LN, LN)]
        hsum[pl.ds(j * LN, LN)] = v
        return 0

    lax.fori_loop(0, nb // LN, col, 0)


def _accum_1d(dst, src, nb):
    def col(j, acc):
        v = dst[pl.ds(j * LN, LN)] + src[pl.ds(j * LN, LN)]
        dst[pl.ds(j * LN, LN)] = v
        return acc + v

    tot = lax.fori_loop(0, nb // LN, col, jnp.zeros((LN,), jnp.int32))
    return jnp.sum(tot)


def _scan_threshold(hsum, nb, target):
    iota = lax.iota(jnp.int32, LN)

    def blk(j, carry):
        found, bsel, csel, run = carry
        v = hsum[pl.ds(j * LN, LN)]
        pref = plsc.cumsum(v)
        cvec = pref + run
        m = cvec >= target
        mi = m.astype(jnp.int32)
        blockhit = jnp.max(mi)
        lane = jnp.min(jnp.where(m, iota, jnp.int32(LN)))
        cblk = jnp.min(jnp.where(m, cvec, IMAX))
        take = (found == 0) & (blockhit == 1)
        bsel = jnp.where(take, j * LN + lane, bsel)
        csel = jnp.where(take, cblk, csel)
        found = jnp.maximum(found, blockhit)
        run = run + jnp.sum(v)
        return found, bsel, csel, run

    found, bsel, csel, _ = lax.fori_loop(
        0, nb // LN, blk,
        (jnp.int32(0), jnp.int32(-1), jnp.int32(0), jnp.int32(0)))
    return bsel, csel


def _chunk_base(g, st, t):
    slab = st * SLABS_PER_TILE + t // CPS
    return (slab * C + g) * HW + (t % CPS) * CH


@functools.partial(
    pl.kernel,
    out_type=[jax.ShapeDtypeStruct((M,), jnp.float32),
              jax.ShapeDtypeStruct((M,), jnp.int32)],
    mesh=_mesh,
    compiler_params=pltpu.CompilerParams(needs_layout_passes=False),
    scratch_types=[pltpu.VMEM((CH,), jnp.float32),
                   pltpu.VMEM((CH,), jnp.float32),
                   pltpu.VMEM((CH,), jnp.int32),
                   pltpu.VMEM((CH,), jnp.int32),
                   pltpu.VMEM((CH,), jnp.int32),
                   pltpu.VMEM((CH,), jnp.int32),
                   pltpu.VMEM((LN * NB1,), jnp.int32),
                   pltpu.VMEM((NB1,), jnp.int32),
                   pltpu.VMEM((NB1,), jnp.int32),
                   pltpu.VMEM((LN,), jnp.int32),
                   pltpu.VMEM_SHARED((NSUB * NB1,), jnp.int32),
                   pltpu.VMEM_SHARED((NSUB * LN,), jnp.int32),
                   pltpu.SemaphoreType.DMA,
                   pltpu.SemaphoreType.DMA,
                   pltpu.SemaphoreType.DMA,
                   pltpu.SemaphoreType.DMA,
                   pltpu.SemaphoreType.DMA,
                   pltpu.SemaphoreType.DMA],
)
def _fused(loss_hbm, tm_hbm, out_hbm, kaug_hbm,
           lbuf0, lbuf1, mbuf0, mbuf1, kbuf0, kbuf1,
           hist, hsum, tmp, cnt, sh_hist, sh_cnt,
           sa0, sa1, sb0, sb1, sc0, sc1):
    ci = lax.axis_index("c")
    si = lax.axis_index("s")
    g = ci * 4 + si // 4
    st = si % 4
    qrow = (si // 4) * 4
    lanes = lax.iota(jnp.int32, LN)
    ones = jnp.ones((LN,), jnp.int32)
    zeros = jnp.zeros((LN,), jnp.int32)

    lbufs, mbufs, kbufs = (lbuf0, lbuf1), (mbuf0, mbuf1), (kbuf0, kbuf1)
    sas, sbs, scs = (sa0, sa1), (sb0, sb1), (sc0, sc1)

    def a_start_in(t, s):
        base = _chunk_base(g, st, t)
        pltpu.async_copy(loss_hbm.at[pl.ds(base, CH)], lbufs[s], sas[s])
        pltpu.async_copy(tm_hbm.at[pl.ds(base, CH)], mbufs[s], sbs[s])

    def a_wait_in(t, s):
        base = _chunk_base(g, st, t)
        pltpu.make_async_copy(loss_hbm.at[pl.ds(base, CH)], lbufs[s],
                              sas[s]).wait()
        pltpu.make_async_copy(tm_hbm.at[pl.ds(base, CH)], mbufs[s],
                              sbs[s]).wait()

    def a_start_out(t, s):
        base = _chunk_base(g, st, t)
        pltpu.async_copy(kbufs[s], kaug_hbm.at[pl.ds(base, CH)], scs[s])

    def a_wait_out(t, s):
        base = _chunk_base(g, st, t)
        pltpu.make_async_copy(kbufs[s], kaug_hbm.at[pl.ds(base, CH)],
                              scs[s]).wait()

    a_start_in(0, 0)
    a_start_in(1, 1)
    _zero_1d(hist, LN * NB1)
    cnt[...] = zeros
    lanebase1 = lanes * NB1

    def a_pair(i2, _):
        for s in range(2):
            t = 2 * i2 + s
            a_wait_in(t, s)

            @pl.when(t >= 2)
            def _():
                a_wait_out(t - 2, s)

            lbuf, mbuf, kbuf = lbufs[s], mbufs[s], kbufs[s]

            @plsc.parallel_loop(0, CH // LN, 1, unroll=U,
                                carry=jnp.zeros((LN,), jnp.int32))
            def csum(i, acc):
                off = i * LN
                lv = lbuf[pl.ds(off, LN)]
                tm = mbuf[pl.ds(off, LN)]
                masked = jnp.where(tm == 0, lv, jnp.float32(0.0))
                key = lax.bitcast_convert_type(masked, jnp.int32)
                kbuf[pl.ds(off, LN)] = jnp.where(tm == 1, IMIN, key)
                bin1 = lax.shift_right_logical(key, 18)
                plsc.addupdate_scatter(hist.at[:], [lanebase1 + bin1], ones,
                                       mask=key != 0)
                return acc + tm

            cnt[...] = cnt[...] + csum
            a_start_out(t, s)

            @pl.when(t + 2 < NCHUNK)
            def _():
                a_start_in(t + 2, s)
        return 0

    lax.fori_loop(0, NCHUNK // 2, a_pair, 0)
    a_wait_out(NCHUNK - 2, 0)
    a_wait_out(NCHUNK - 1, 1)

    _merge_lanes(hist, hsum, NB1)
    pltpu.sync_copy(hsum, sh_hist.at[pl.ds(si * NB1, NB1)])
    pltpu.sync_copy(cnt, sh_cnt.at[pl.ds(si * LN, LN)])
    plsc.subcore_barrier()

    def k_start_in(t, s):
        base = _chunk_base(g, st, t)
        pltpu.async_copy(kaug_hbm.at[pl.ds(base, CH)], kbufs[s], sas[s])

    def k_wait_in(t, s):
        base = _chunk_base(g, st, t)
        pltpu.make_async_copy(kaug_hbm.at[pl.ds(base, CH)], kbufs[s],
                              sas[s]).wait()

    def k_stream(vec_fn):
        def pair(i2, _):
            for s in range(2):
                t = 2 * i2 + s
                k_wait_in(t, s)

                @plsc.parallel_loop(0, CH // LN, 1, unroll=U)
                def _vec(i):
                    vec_fn(kbufs[s], i * LN)

                @pl.when(t + 2 < NCHUNK)
                def _():
                    k_start_in(t + 2, s)
            return 0

        lax.fori_loop(0, NCHUNK // 2, pair, 0)

    k_start_in(0, 0)
    k_start_in(1, 1)

    pltpu.sync_copy(sh_hist.at[pl.ds(qrow * NB1, NB1)], hsum)
    total = jnp.int32(0)
    for r in range(1, 4):
        pltpu.sync_copy(sh_hist.at[pl.ds((qrow + r) * NB1, NB1)], tmp)
        total = _accum_1d(hsum, tmp, NB1)
    pltpu.sync_copy(sh_cnt.at[pl.ds(qrow * LN, 4 * LN)],
                    hist.at[pl.ds(0, 4 * LN)])
    cntpos = jnp.sum(hist[pl.ds(0, LN)] + hist[pl.ds(LN, LN)]
                     + hist[pl.ds(2 * LN, LN)] + hist[pl.ds(3 * LN, LN)])
    k = jnp.minimum(jnp.maximum(cntpos, 1), total)
    target = total - k + 1
    b1, cb1 = _scan_threshold(hsum, NB1, target)
    r1 = k - (total - cb1)
    valid = (total > 0).astype(jnp.int32)
    plsc.subcore_barrier()

    _zero_1d(hist, LN * NB2)
    b1v = jnp.full((LN,), b1, jnp.int32)
    lanebase2 = lanes * NB2

    def c_vec(kbuf, off):
        kaug = kbuf[pl.ds(off, LN)]
        m = (lax.shift_right_logical(kaug, 18) == b1v) & (kaug != 0)
        bin2 = lax.shift_right_logical(kaug, 6) & jnp.int32(0xFFF)
        plsc.addupdate_scatter(hist.at[:], [lanebase2 + bin2], ones, mask=m)

    k_stream(c_vec)

    _merge_lanes(hist, hsum, NB2)
    pltpu.sync_copy(hsum, sh_hist.at[pl.ds(si * NB1, NB2)])
    plsc.subcore_barrier()

    k_start_in(0, 0)
    k_start_in(1, 1)

    pltpu.sync_copy(sh_hist.at[pl.ds(qrow * NB1, NB2)], hsum)
    total2 = jnp.int32(0)
    for r in range(1, 4):
        pltpu.sync_copy(sh_hist.at[pl.ds((qrow + r) * NB1, NB2)], tmp)
        total2 = _accum_1d(hsum, tmp, NB2)
    target2 = total2 - r1 + 1
    b2, cb2 = _scan_threshold(hsum, NB2, target2)
    r2 = r1 - (total2 - cb2)
    b12 = lax.shift_left(b1, 12) | b2
    plsc.subcore_barrier()

    _zero_1d(hist, LN * NB3)
    b12v = jnp.full((LN,), b12, jnp.int32)
    lanebase3 = lanes * NB3

    def e_vec(kbuf, off):
        kaug = kbuf[pl.ds(off, LN)]
        m = (lax.shift_right_logical(kaug, 6) == b12v) & (kaug != 0)
        bin3 = kaug & jnp.int32(0x3F)
        plsc.addupdate_scatter(hist.at[:], [lanebase3 + bin3], ones, mask=m)

    k_stream(e_vec)

    _merge_lanes(hist, hsum, NB3)
    pltpu.sync_copy(hsum.at[pl.ds(0, NB3)], sh_hist.at[pl.ds(si * NB1, NB3)])
    plsc.subcore_barrier()

    k_start_in(0, 0)
    k_start_in(1, 1)

    pltpu.sync_copy(sh_hist.at[pl.ds(qrow * NB1, NB3)], hsum.at[pl.ds(0, NB3)])
    total3 = jnp.int32(0)
    for r in range(1, 4):
        pltpu.sync_copy(sh_hist.at[pl.ds((qrow + r) * NB1, NB3)],
                        tmp.at[pl.ds(0, NB3)])
        total3 = _accum_1d(hsum, tmp, NB3)
    target3 = total3 - r2 + 1
    b3, _cb3 = _scan_threshold(hsum, NB3, target3)
    t_key = lax.shift_left(b12, 6) | b3
    t_key = jnp.where(valid == 1, t_key, jnp.int32(2 ** 30))
    txor = jnp.full((LN,), t_key ^ IMIN, jnp.int32)

    obufs = (lbuf0, lbuf1)

    def g_start_out(t, s):
        base = _chunk_base(g, st, t)
        pltpu.async_copy(obufs[s], out_hbm.at[pl.ds(base, CH)], sbs[s])

    def g_wait_out(t, s):
        base = _chunk_base(g, st, t)
        pltpu.make_async_copy(obufs[s], out_hbm.at[pl.ds(base, CH)],
                              sbs[s]).wait()

    def g_pair(i2, _):
        for s in range(2):
            t = 2 * i2 + s
            k_wait_in(t, s)

            @pl.when(t >= 2)
            def _():
                g_wait_out(t - 2, s)

            kbuf, obuf = kbufs[s], obufs[s]

            @plsc.parallel_loop(0, CH // LN, 1, unroll=U)
            def _vec(i):
                off = i * LN
                kaug = kbuf[pl.ds(off, LN)]
                sel = (kaug ^ IMIN) >= txor
                obuf[pl.ds(off, LN)] = jnp.where(sel, jnp.float32(1.0),
                                                 jnp.float32(0.0))

            g_start_out(t, s)

            @pl.when(t + 2 < NCHUNK)
            def _():
                k_start_in(t + 2, s)
        return 0

    lax.fori_loop(0, NCHUNK // 2, g_pair, 0)
    g_wait_out(NCHUNK - 2, 0)
    g_wait_out(NCHUNK - 1, 1)


def kernel(loss, type_mask):
    lf = loss.reshape(M)
    tf = type_mask.reshape(M)
    out, _ = _fused(lf, tf)
    return out.reshape(loss.shape)
